# Initial kernel scaffold; baseline (speedup 1.0000x reference)
import jax
import jax.numpy as jnp
from jax.experimental import pallas as pl


def kernel(x, edge_index, y_label, target_mask, params):
    raise NotImplementedError("write your pallas kernel here")


# speedup vs baseline: 1.0614x; 1.0614x over previous
import functools

import jax
import jax.numpy as jnp
from jax.experimental import pallas as pl
from jax.experimental.pallas import tpu as pltpu

N = 10000
E = 320000
D = 128
HID = 16
HEADS = 8
TH = 128
PROJ = 256
OUT = 64
HOPS = 2
TAU = 0.8


def _bn(x, g, b):
    m = x.mean(0)
    v = x.var(0)
    return g * (x - m) / jnp.sqrt(v + 1e-5) + b


def _gat(x, src, dst, W, a_s, a_d, bias):
    n = x.shape[0]
    h = (x @ W).reshape(n, HEADS, HID)
    s_src = (h * a_s).sum(-1)
    s_dst = (h * a_d).sum(-1)
    Kraw = s_src.max(0) + s_dst.max(0)
    K = jnp.where(Kraw > 0, Kraw, 0.2 * Kraw)
    al = s_src[src] + s_dst[dst]
    al = jnp.where(al > 0, al, 0.2 * al)
    ex = jnp.exp(al - K)
    S = jax.ops.segment_sum(ex, dst, n)
    U = jax.ops.segment_sum(h[src] * ex[:, :, None], dst, n)
    out = U / (S + 1e-16)[:, :, None]
    return out.reshape(n, TH) + bias


def _gumbel_pallas_body(lr_ref, g0_ref, g1_ref, smask_ref):
    lr = lr_ref[...]
    a0 = g0_ref[...] / TAU
    a1 = (lr + g1_ref[...]) / TAU
    m = jnp.maximum(a0, a1)
    e0 = jnp.exp(a0 - m)
    e1 = jnp.exp(a1 - m)
    ssum = e0 + e1
    soft1 = e1 / ssum
    soft0 = e0 / ssum
    hard1 = (soft1 > soft0).astype(jnp.float32)
    smask_ref[...] = (hard1 - soft1) + soft1


def _gumbel_smask(logits_raw, gumbel):
    R, C = E // 128, 128
    out = pl.pallas_call(
        _gumbel_pallas_body,
        out_shape=jax.ShapeDtypeStruct((R, C), jnp.float32),
    )(logits_raw.reshape(R, C),
      gumbel[:, 0].reshape(R, C),
      gumbel[:, 1].reshape(R, C))
    return out.reshape(E)


def kernel(x, edge_index, y_label, target_mask, params):
    p = params
    gumbel = jax.random.gumbel(jax.random.key(123), (E, 2), jnp.float32)
    row, col = edge_index[0], edge_index[1]
    sl = jnp.arange(N, dtype=row.dtype)
    src = jnp.concatenate([row, sl])
    dst = jnp.concatenate([col, sl])
    x_proj = x @ p['W_res'] + p['b_res']
    y_emb = p['emb'][y_label]
    x_fused = x_proj + y_emb
    h1 = jax.nn.elu(_bn(_gat(x_fused, src, dst, p['W_g1'], p['as1'], p['ad1'], p['bg1']), p['g1_g'], p['g1_b']) + y_emb)
    h_base = jax.nn.elu(_bn(_gat(h1, src, dst, p['W_g2'], p['as2'], p['ad2'], p['bg2']), p['g2_g'], p['g2_b']) + x_proj)
    P = h_base @ p['Wp'] + p['bp']
    Q = P @ p['Wbil']
    logits_raw = jnp.clip((Q[row] * P[col]).sum(-1) + p['bias_s'][0], -10.0, 10.0)
    smask = _gumbel_smask(logits_raw, gumbel)
    active = target_mask.astype(jnp.float32)
    fw = jnp.zeros((E,), jnp.float32)
    for _ in range(HOPS):
        cur = smask * active[col]
        fw = jnp.maximum(fw, cur)
        na = jnp.zeros((N,), jnp.float32).at[row].add(cur)
        active = (na > 1e-5).astype(jnp.float32)
    ex = jnp.exp(fw)
    s = jax.ops.segment_sum(ex, col, N)
    nw = ex / (s[col] + 1e-16)
    h = h_base
    for _ in range(HOPS):
        xl = h @ p['W_enh'] + p['b_enh']
        agg = jax.ops.segment_sum(nw[:, None] * xl[row], col, N)
        h = jax.nn.elu(_bn(h + agg, p['enh_g'], p['enh_b']))
    c = h @ p['Wc1'] + p['bc1']
    c = jax.nn.relu(_bn(c, p['c_g'], p['c_b']))
    logits = c @ p['Wc2'] + p['bc2']
    logp = jax.nn.log_softmax(logits, -1)
    A = h @ p['Wl1'][:TH] + p['bl1']
    B = h @ p['Wl1'][TH:]
    lk = jax.nn.relu(A[row] + B[col]) @ p['Wl2'] + p['bl2']
    lk = jax.nn.sigmoid(lk)[:, 0]
    return logp, lk, h, fw, jax.lax.stop_gradient(logits_raw)


# speedup vs baseline: 21.2963x; 20.0637x over previous
import functools

import jax
import jax.numpy as jnp
from jax import lax
from jax.experimental import pallas as pl
from jax.experimental.pallas import tpu as pltpu
from jax.experimental.pallas import tpu_sc as plsc

N = 10000
E = 320000
D = 128
HID = 16
HEADS = 8
TH = 128
PROJ = 256
OUT = 64
HOPS = 2
TAU = 0.8

NC = 2
NS = 16
NW = NC * NS
L = 16

BLK = 128
SBLK = 2048
EP = 327680
EPS = 331776
BLKG = 96
NP = 10240
ROWS_PER = NP // NS


def _mesh():
    return plsc.VectorSubcoreMesh(
        core_axis_name="c", subcore_axis_name="s", num_cores=NC,
        num_subcores=NS)


_SC_PARAMS = pltpu.CompilerParams(
    use_tc_tiling_on_sc=False, needs_layout_passes=False)


def _wid():
    return lax.axis_index("s") * NC + lax.axis_index("c")


def _lane_valid(gid_base, e_actual):
    gid = gid_base + lax.iota(jnp.int32, L)
    return gid < e_actual


def _make_gat_pass(nb, e_actual):
    @functools.partial(
        pl.kernel,
        out_type=jax.ShapeDtypeStruct((NC * NP, 144), jnp.float32),
        mesh=_mesh(),
        compiler_params=_SC_PARAMS,
        scratch_types=[
            pltpu.VMEM((BLKG,), jnp.int32),
            pltpu.VMEM((BLKG,), jnp.int32),
            pltpu.VMEM((BLKG, 144), jnp.float32),
            pltpu.VMEM((BLKG, 16), jnp.float32),
            pltpu.VMEM((BLKG, 144), jnp.float32),
            pltpu.VMEM((16,), jnp.float32),
            pltpu.VMEM((16,), jnp.float32),
            pltpu.VMEM_SHARED((NP, 144), jnp.float32),
            pltpu.SemaphoreType.DMA,
            pltpu.SemaphoreType.DMA,
        ],
    )
    def gat_pass(src_hbm, dst_hbm, tabs_hbm, tabd_hbm, kvec_hbm, zeros_hbm,
                 out_hbm, src_v, dst_v, rows_v, rowd_v, msg_v, kv_v, ex_v,
                 acc, sem1, sem2):
        c = lax.axis_index("c")
        s = lax.axis_index("s")
        wid = s * NC + c
        pltpu.sync_copy(kvec_hbm, kv_v)
        pltpu.sync_copy(zeros_hbm, acc.at[pl.ds(s * ROWS_PER, ROWS_PER)])
        plsc.subcore_barrier()

        def block(b, _):
            base = (wid * nb + b) * BLKG
            pltpu.sync_copy(src_hbm.at[pl.ds(base, BLKG)], src_v)
            pltpu.sync_copy(dst_hbm.at[pl.ds(base, BLKG)], dst_v)
            cp1 = pltpu.async_copy(tabs_hbm.at[src_v], rows_v, sem1)
            cp2 = pltpu.async_copy(tabd_hbm.at[dst_v], rowd_v, sem2)
            cp1.wait()
            cp2.wait()
            kvv = kv_v[...]

            def edge(e, _):
                sv = rows_v[e, pl.ds(128, 16)]
                dv = rowd_v[e, :]
                al = sv + dv
                al = jnp.where(al > 0, al, 0.2 * al)
                exv = jnp.exp(al - kvv)
                valid = ((base + e) < e_actual).astype(jnp.float32)
                exv = exv * valid
                msg_v[e, pl.ds(128, 16)] = exv
                for h in range(HEADS):
                    bc = jnp.broadcast_to(exv[h], (L,))
                    msg_v[e, pl.ds(h * 16, 16)] = bc * rows_v[e, pl.ds(h * 16, 16)]
                return 0

            lax.fori_loop(0, BLKG, edge, 0)
            pltpu.sync_copy(msg_v, acc.at[dst_v], add=True)
            return 0

        lax.fori_loop(0, nb, block, 0)
        plsc.subcore_barrier()
        off = c * NP + s * ROWS_PER
        pltpu.sync_copy(acc.at[pl.ds(s * ROWS_PER, ROWS_PER)],
                        out_hbm.at[pl.ds(off, ROWS_PER)])

    return gat_pass


def _make_dot_pass(nb, width):
    nv = width // L

    @functools.partial(
        pl.kernel,
        out_type=jax.ShapeDtypeStruct((EP,), jnp.float32),
        mesh=_mesh(),
        compiler_params=_SC_PARAMS,
        scratch_types=[
            pltpu.VMEM((BLK,), jnp.int32),
            pltpu.VMEM((BLK,), jnp.int32),
            pltpu.VMEM((BLK, width), jnp.float32),
            pltpu.VMEM((BLK, width), jnp.float32),
            pltpu.VMEM((BLK,), jnp.float32),
            pltpu.SemaphoreType.DMA,
            pltpu.SemaphoreType.DMA,
        ],
    )
    def dot_pass(row_hbm, col_hbm, qtab_hbm, ptab_hbm, out_hbm,
                 row_v, col_v, qa_v, pb_v, out_v, sem1, sem2):
        wid = _wid()

        def block(b, _):
            base = (wid * nb + b) * BLK
            pltpu.sync_copy(row_hbm.at[pl.ds(base, BLK)], row_v)
            pltpu.sync_copy(col_hbm.at[pl.ds(base, BLK)], col_v)
            cp1 = pltpu.async_copy(qtab_hbm.at[row_v], qa_v, sem1)
            cp2 = pltpu.async_copy(ptab_hbm.at[col_v], pb_v, sem2)
            cp1.wait()
            cp2.wait()

            lane = lax.iota(jnp.int32, L)

            def grp(g, _):
                res = jnp.zeros((L,), jnp.float32)
                for j in range(L):
                    e = g * L + j
                    acc = qa_v[e, pl.ds(0, 16)] * pb_v[e, pl.ds(0, 16)]
                    for k in range(1, nv):
                        acc = acc + qa_v[e, pl.ds(k * 16, 16)] * pb_v[e, pl.ds(k * 16, 16)]
                    res = jnp.where(lane == j, jnp.sum(acc), res)
                out_v[pl.ds(g * L, L)] = res
                return 0

            lax.fori_loop(0, BLK // L, grp, 0)
            pltpu.sync_copy(out_v, out_hbm.at[pl.ds(base, BLK)])
            return 0

        lax.fori_loop(0, nb, block, 0)

    return dot_pass


def _make_lk_pass(nb):
    @functools.partial(
        pl.kernel,
        out_type=jax.ShapeDtypeStruct((EP,), jnp.float32),
        mesh=_mesh(),
        compiler_params=_SC_PARAMS,
        scratch_types=[
            pltpu.VMEM((BLK,), jnp.int32),
            pltpu.VMEM((BLK,), jnp.int32),
            pltpu.VMEM((BLK, 64), jnp.float32),
            pltpu.VMEM((BLK, 64), jnp.float32),
            pltpu.VMEM((64,), jnp.float32),
            pltpu.VMEM((BLK,), jnp.float32),
            pltpu.SemaphoreType.DMA,
            pltpu.SemaphoreType.DMA,
        ],
    )
    def lk_pass(row_hbm, col_hbm, atab_hbm, btab_hbm, w_hbm, out_hbm,
                row_v, col_v, a_v, b_v, w_v, out_v, sem1, sem2):
        wid = _wid()
        pltpu.sync_copy(w_hbm, w_v)
        wr = [w_v[pl.ds(k * 16, 16)] for k in range(4)]

        def block(b, _):
            base = (wid * nb + b) * BLK
            pltpu.sync_copy(row_hbm.at[pl.ds(base, BLK)], row_v)
            pltpu.sync_copy(col_hbm.at[pl.ds(base, BLK)], col_v)
            cp1 = pltpu.async_copy(atab_hbm.at[row_v], a_v, sem1)
            cp2 = pltpu.async_copy(btab_hbm.at[col_v], b_v, sem2)
            cp1.wait()
            cp2.wait()

            lane = lax.iota(jnp.int32, L)

            def grp(g, _):
                res = jnp.zeros((L,), jnp.float32)
                for j in range(L):
                    e = g * L + j
                    t = jnp.maximum(a_v[e, pl.ds(0, 16)] + b_v[e, pl.ds(0, 16)], 0.0)
                    acc = t * wr[0]
                    for k in range(1, 4):
                        t = jnp.maximum(
                            a_v[e, pl.ds(k * 16, 16)] + b_v[e, pl.ds(k * 16, 16)], 0.0)
                        acc = acc + t * wr[k]
                    res = jnp.where(lane == j, jnp.sum(acc), res)
                out_v[pl.ds(g * L, L)] = res
                return 0

            lax.fori_loop(0, BLK // L, grp, 0)
            pltpu.sync_copy(out_v, out_hbm.at[pl.ds(base, BLK)])
            return 0

        lax.fori_loop(0, nb, block, 0)

    return lk_pass


def _make_hop1(nbs):
    @functools.partial(
        pl.kernel,
        out_type=(jax.ShapeDtypeStruct((EP,), jnp.float32),
                  jax.ShapeDtypeStruct((NW * N,), jnp.float32)),
        mesh=_mesh(),
        compiler_params=_SC_PARAMS,
        scratch_types=[
            pltpu.VMEM((SBLK,), jnp.int32),
            pltpu.VMEM((SBLK,), jnp.int32),
            pltpu.VMEM((SBLK,), jnp.float32),
            pltpu.VMEM((SBLK,), jnp.float32),
            pltpu.VMEM((N,), jnp.float32),
            pltpu.VMEM((N,), jnp.float32),
        ],
    )
    def hop1(row_hbm, col_hbm, sm_hbm, act_hbm, fw_hbm, na_hbm,
             row_v, col_v, sm_v, fw_v, act_v, nal_v):
        wid = _wid()
        pltpu.sync_copy(act_hbm, act_v)

        def zero(i, _):
            nal_v[pl.ds(i * L, L)] = jnp.zeros((L,), jnp.float32)
            return 0

        lax.fori_loop(0, N // L, zero, 0)

        def block(b, _):
            base = (wid * nbs + b) * SBLK
            pltpu.sync_copy(row_hbm.at[pl.ds(base, SBLK)], row_v)
            pltpu.sync_copy(col_hbm.at[pl.ds(base, SBLK)], col_v)
            pltpu.sync_copy(sm_hbm.at[pl.ds(base, SBLK)], sm_v)

            def grp(i, _):
                col16 = col_v[pl.ds(i * L, L)]
                av = plsc.load_gather(act_v, [col16])
                cur = sm_v[pl.ds(i * L, L)] * av
                fw_v[pl.ds(i * L, L)] = cur
                row16 = row_v[pl.ds(i * L, L)]
                plsc.addupdate_scatter(nal_v, [row16], cur)
                return 0

            lax.fori_loop(0, SBLK // L, grp, 0)
            pltpu.sync_copy(fw_v, fw_hbm.at[pl.ds(base, SBLK)])
            return 0

        lax.fori_loop(0, nbs, block, 0)
        pltpu.sync_copy(nal_v, na_hbm.at[pl.ds(wid * N, N)])

    return hop1


def _make_hop2(nbs, e_actual):
    @functools.partial(
        pl.kernel,
        out_type=(jax.ShapeDtypeStruct((EP,), jnp.float32),
                  jax.ShapeDtypeStruct((NW * N,), jnp.float32)),
        mesh=_mesh(),
        compiler_params=_SC_PARAMS,
        scratch_types=[
            pltpu.VMEM((SBLK,), jnp.int32),
            pltpu.VMEM((SBLK,), jnp.float32),
            pltpu.VMEM((SBLK,), jnp.float32),
            pltpu.VMEM((SBLK,), jnp.float32),
            pltpu.VMEM((N,), jnp.float32),
            pltpu.VMEM((N,), jnp.float32),
        ],
    )
    def hop2(col_hbm, sm_hbm, act_hbm, fw1_hbm, fw_hbm, s_hbm,
             col_v, sm_v, fw1_v, fw_v, act_v, sl_v):
        wid = _wid()
        pltpu.sync_copy(act_hbm, act_v)

        def zero(i, _):
            sl_v[pl.ds(i * L, L)] = jnp.zeros((L,), jnp.float32)
            return 0

        lax.fori_loop(0, N // L, zero, 0)

        def block(b, _):
            base = (wid * nbs + b) * SBLK
            pltpu.sync_copy(col_hbm.at[pl.ds(base, SBLK)], col_v)
            pltpu.sync_copy(sm_hbm.at[pl.ds(base, SBLK)], sm_v)
            pltpu.sync_copy(fw1_hbm.at[pl.ds(base, SBLK)], fw1_v)

            def grp(i, _):
                col16 = col_v[pl.ds(i * L, L)]
                av = plsc.load_gather(act_v, [col16])
                cur = sm_v[pl.ds(i * L, L)] * av
                fwv = jnp.maximum(fw1_v[pl.ds(i * L, L)], cur)
                fw_v[pl.ds(i * L, L)] = fwv
                exv = jnp.exp(fwv)
                exv = jnp.where(_lane_valid(base + i * L, e_actual), exv, 0.0)
                plsc.addupdate_scatter(sl_v, [col16], exv)
                return 0

            lax.fori_loop(0, SBLK // L, grp, 0)
            pltpu.sync_copy(fw_v, fw_hbm.at[pl.ds(base, SBLK)])
            return 0

        lax.fori_loop(0, nbs, block, 0)
        pltpu.sync_copy(sl_v, s_hbm.at[pl.ds(wid * N, N)])

    return hop2


def _make_enh_pass(nb, e_actual):
    @functools.partial(
        pl.kernel,
        out_type=jax.ShapeDtypeStruct((NC * NP, TH), jnp.float32),
        mesh=_mesh(),
        compiler_params=_SC_PARAMS,
        scratch_types=[
            pltpu.VMEM((BLK,), jnp.int32),
            pltpu.VMEM((BLK,), jnp.int32),
            pltpu.VMEM((BLK, TH), jnp.float32),
            pltpu.VMEM((BLK, TH), jnp.float32),
            pltpu.VMEM((BLK,), jnp.float32),
            pltpu.VMEM((BLK,), jnp.float32),
            pltpu.VMEM((N,), jnp.float32),
            pltpu.VMEM_SHARED((NP, TH), jnp.float32),
            pltpu.SemaphoreType.DMA,
            pltpu.SemaphoreType.DMA,
        ],
    )
    def enh_pass(row_hbm, col_hbm, fw_hbm, sp_hbm, xl_hbm, zeros_hbm,
                 out_hbm, row_v, col_v, rows_v, msg_v, fw_v, nw_v, s_v,
                 acc, sem1, sem2):
        c = lax.axis_index("c")
        s = lax.axis_index("s")
        wid = s * NC + c
        pltpu.sync_copy(sp_hbm, s_v)
        pltpu.sync_copy(zeros_hbm, acc.at[pl.ds(s * ROWS_PER, ROWS_PER)])
        plsc.subcore_barrier()

        def block(b, _):
            base = (wid * nb + b) * BLK
            pltpu.sync_copy(row_hbm.at[pl.ds(base, BLK)], row_v)
            pltpu.sync_copy(col_hbm.at[pl.ds(base, BLK)], col_v)
            pltpu.sync_copy(fw_hbm.at[pl.ds(base, BLK)], fw_v)
            cp1 = pltpu.async_copy(xl_hbm.at[row_v], rows_v, sem1)
            cp1.wait()

            def grp(g, _):
                fw16 = fw_v[pl.ds(g * L, L)]
                col16 = col_v[pl.ds(g * L, L)]
                sg = plsc.load_gather(s_v, [col16])
                nwv = jnp.exp(fw16) / sg
                nwv = jnp.where(_lane_valid(base + g * L, e_actual), nwv, 0.0)
                for j in range(L):
                    e = g * L + j
                    bc = jnp.broadcast_to(nwv[j], (L,))
                    for h in range(TH // L):
                        msg_v[e, pl.ds(h * 16, 16)] = bc * rows_v[e, pl.ds(h * 16, 16)]
                return 0

            lax.fori_loop(0, BLK // L, grp, 0)
            pltpu.sync_copy(msg_v, acc.at[col_v], add=True)
            return 0

        lax.fori_loop(0, nb, block, 0)
        plsc.subcore_barrier()
        off = c * NP + s * ROWS_PER
        pltpu.sync_copy(acc.at[pl.ds(s * ROWS_PER, ROWS_PER)],
                        out_hbm.at[pl.ds(off, ROWS_PER)])

    return enh_pass


_gat_pass = _make_gat_pass(EPS // (NW * BLKG), E + N)
_dot_pass = _make_dot_pass(EP // (NW * BLK), PROJ)
_lk_pass = _make_lk_pass(EP // (NW * BLK))
_hop1 = _make_hop1(EP // (NW * SBLK))
_hop2 = _make_hop2(EP // (NW * SBLK), E)
_enh_pass = _make_enh_pass(EP // (NW * BLK), E)


def _gumbel_pallas_body(lr_ref, g0_ref, g1_ref, smask_ref):
    lr = lr_ref[...]
    a0 = g0_ref[...] / TAU
    a1 = (lr + g1_ref[...]) / TAU
    m = jnp.maximum(a0, a1)
    e0 = jnp.exp(a0 - m)
    e1 = jnp.exp(a1 - m)
    ssum = e0 + e1
    soft1 = e1 / ssum
    soft0 = e0 / ssum
    hard1 = (soft1 > soft0).astype(jnp.float32)
    smask_ref[...] = (hard1 - soft1) + soft1


def _gumbel_smask(logits_raw, gumbel):
    R, C = E // 128, 128
    out = pl.pallas_call(
        _gumbel_pallas_body,
        out_shape=jax.ShapeDtypeStruct((R, C), jnp.float32),
    )(logits_raw.reshape(R, C),
      gumbel[:, 0].reshape(R, C),
      gumbel[:, 1].reshape(R, C))
    return out.reshape(E)


def _bn(x, g, b):
    m = x.mean(0)
    v = x.var(0)
    return g * (x - m) / jnp.sqrt(v + 1e-5) + b


def _gat_layer(x, srcp, dstp, W, a_s, a_d, bias, zeros144):
    h = (x @ W).reshape(N, HEADS, HID)
    s_src = (h * a_s).sum(-1)
    s_dst = (h * a_d).sum(-1)
    Kraw = s_src.max(0) + s_dst.max(0)
    K = jnp.where(Kraw > 0, Kraw, 0.2 * Kraw)
    kvec = jnp.concatenate([K, jnp.zeros((8,), jnp.float32)])
    hm = h.reshape(N, TH)
    tabs = jnp.concatenate([hm, s_src, jnp.zeros((N, 8), jnp.float32)], axis=1)
    tabd = jnp.concatenate([s_dst, jnp.zeros((N, 8), jnp.float32)], axis=1)
    part = _gat_pass(srcp, dstp, tabs, tabd, kvec, zeros144)
    u2 = part[:N] + part[NP:NP + N]
    S = u2[:, 128:136]
    out = u2[:, :128].reshape(N, HEADS, HID) / (S + 1e-16)[:, :, None]
    return out.reshape(N, TH) + bias


def kernel(x, edge_index, y_label, target_mask, params):
    p = params
    gumbel = jax.random.gumbel(jax.random.key(123), (E, 2), jnp.float32)
    row, col = edge_index[0], edge_index[1]
    sl = jnp.arange(N, dtype=row.dtype)
    rowp = jnp.concatenate([row, jnp.zeros((EP - E,), row.dtype)])
    colp = jnp.concatenate([col, jnp.zeros((EP - E,), col.dtype)])
    srcp = jnp.concatenate([row, sl, jnp.zeros((EPS - E - N,), row.dtype)])
    dstp = jnp.concatenate([col, sl, jnp.zeros((EPS - E - N,), col.dtype)])
    zeros144 = jnp.zeros((ROWS_PER, 144), jnp.float32)
    zeros128 = jnp.zeros((ROWS_PER, TH), jnp.float32)

    x_proj = x @ p['W_res'] + p['b_res']
    y_emb = p['emb'][y_label]
    x_fused = x_proj + y_emb
    h1 = jax.nn.elu(_bn(_gat_layer(x_fused, srcp, dstp, p['W_g1'], p['as1'],
                                   p['ad1'], p['bg1'], zeros144),
                        p['g1_g'], p['g1_b']) + y_emb)
    h_base = jax.nn.elu(_bn(_gat_layer(h1, srcp, dstp, p['W_g2'], p['as2'],
                                       p['ad2'], p['bg2'], zeros144),
                            p['g2_g'], p['g2_b']) + x_proj)

    P = h_base @ p['Wp'] + p['bp']
    Q = P @ p['Wbil']
    dots = _dot_pass(rowp, colp, Q, P)
    logits_raw = jnp.clip(dots[:E] + p['bias_s'][0], -10.0, 10.0)
    smask = _gumbel_smask(logits_raw, gumbel)
    smaskp = jnp.concatenate([smask, jnp.zeros((EP - E,), jnp.float32)])

    active0 = target_mask.astype(jnp.float32)
    fw1p, na_part = _hop1(rowp, colp, smaskp, active0)
    active1 = (na_part.reshape(NW, N).sum(0) > 1e-5).astype(jnp.float32)
    fwp, s_part = _hop2(colp, smaskp, active1, fw1p)
    fw = fwp[:E]
    s_plus = s_part.reshape(NW, N).sum(0) + 1e-16

    h = h_base
    for _ in range(HOPS):
        xl = h @ p['W_enh'] + p['b_enh']
        part = _enh_pass(rowp, colp, fwp, s_plus, xl, zeros128)
        agg = part[:N] + part[NP:NP + N]
        h = jax.nn.elu(_bn(h + agg, p['enh_g'], p['enh_b']))

    c = h @ p['Wc1'] + p['bc1']
    c = jax.nn.relu(_bn(c, p['c_g'], p['c_b']))
    logits = c @ p['Wc2'] + p['bc2']
    logp = jax.nn.log_softmax(logits, -1)

    A = h @ p['Wl1'][:TH] + p['bl1']
    B = h @ p['Wl1'][TH:]
    lkdots = _lk_pass(rowp, colp, A, B, p['Wl2'][:, 0])
    lk = jax.nn.sigmoid(lkdots[:E] + p['bl2'][0])
    return logp, lk, h, fw, jax.lax.stop_gradient(logits_raw)
